# Initial kernel scaffold; baseline (speedup 1.0000x reference)
import jax
import jax.numpy as jnp
from jax.experimental import pallas as pl


def kernel(x, edge_index, W1, att_src1, att_dst1, b1, W2, att_src2, att_dst2, b2, Wo, bo):
    raise NotImplementedError("write your pallas kernel here")


# speedup vs baseline: 30.4589x; 30.4589x over previous
import functools

import jax
import jax.numpy as jnp
from jax import lax
from jax.experimental import pallas as pl
from jax.experimental.pallas import tpu as pltpu
from jax.experimental.pallas import tpu_sc as plsc

N_NODES = 10000
D_IN = 128
HID = 64
HEADS = 3
F = HEADS * HID
FH = 96
FP = FH + 16
N_CLASSES = 40

NC = 2
NS = 16
CHUNK = 128
NP = 10240
RPT = NP // NS
NZCH = RPT // CHUNK

E_TOT = 330000
NCHUNK = -(-E_TOT // (NS * CHUNK))
EPAD = NS * NCHUNK * CHUNK

BR = 1000

HEAD_MAP = ((0, 0, 0, 0, 1, 1), (1, 1, 2, 2, 2, 2))


def _proj_body(x_ref, w0_ref, w1_ref, wa_ref, hst_ref, atab_ref):
    x = x_ref[...]
    hst_ref[0] = jnp.dot(x, w0_ref[...], preferred_element_type=jnp.float32)
    hst_ref[1] = jnp.dot(x, w1_ref[...], preferred_element_type=jnp.float32)
    atab_ref[...] = jnp.dot(x, wa_ref[...], preferred_element_type=jnp.float32)


def _proj(x, wb0, wb1, wa):
    k = x.shape[1]
    return pl.pallas_call(
        _proj_body,
        grid=(N_NODES // BR,),
        in_specs=[
            pl.BlockSpec((BR, k), lambda i: (i, 0)),
            pl.BlockSpec((k, FP), lambda i: (0, 0)),
            pl.BlockSpec((k, FP), lambda i: (0, 0)),
            pl.BlockSpec((k, 16), lambda i: (0, 0)),
        ],
        out_specs=[
            pl.BlockSpec((2, BR, FP), lambda i: (0, i, 0)),
            pl.BlockSpec((BR, 16), lambda i: (i, 0)),
        ],
        out_shape=[
            jax.ShapeDtypeStruct((2, N_NODES, FP), jnp.float32),
            jax.ShapeDtypeStruct((N_NODES, 16), jnp.float32),
        ],
    )(x, wb0, wb1, wa)


def _normalize(acc_ref, sel_ref, eh_ref, b_ref):
    val0 = acc_ref[0]
    val1 = acc_ref[1]
    msg = jnp.concatenate([val0[:, :FH], val1[:, :FH]], axis=1)
    den = jnp.dot(val0, sel_ref[...], preferred_element_type=jnp.float32)
    dfull = jnp.dot(1.0 / (den + 1e-16), eh_ref[...],
                    preferred_element_type=jnp.float32)
    return jnp.maximum(msg * dfull + b_ref[...], 0.0)


def _combine_body(acc_ref, sel_ref, eh_ref, b_ref, w0_ref, w1_ref, wa_ref,
                  hst_ref, atab_ref):
    h1 = _normalize(acc_ref, sel_ref, eh_ref, b_ref)
    hst_ref[0] = jnp.dot(h1, w0_ref[...], preferred_element_type=jnp.float32)
    hst_ref[1] = jnp.dot(h1, w1_ref[...], preferred_element_type=jnp.float32)
    atab_ref[...] = jnp.dot(h1, wa_ref[...],
                            preferred_element_type=jnp.float32)


def _combine_proj(acc, sel, ehpad, bpad, wb0, wb1, wa):
    return pl.pallas_call(
        _combine_body,
        grid=(N_NODES // BR,),
        in_specs=[
            pl.BlockSpec((2, BR, FP), lambda i: (0, i, 0)),
            pl.BlockSpec((FP, HEADS), lambda i: (0, 0)),
            pl.BlockSpec((HEADS, F), lambda i: (0, 0)),
            pl.BlockSpec((1, F), lambda i: (0, 0)),
            pl.BlockSpec((F, FP), lambda i: (0, 0)),
            pl.BlockSpec((F, FP), lambda i: (0, 0)),
            pl.BlockSpec((F, 16), lambda i: (0, 0)),
        ],
        out_specs=[
            pl.BlockSpec((2, BR, FP), lambda i: (0, i, 0)),
            pl.BlockSpec((BR, 16), lambda i: (i, 0)),
        ],
        out_shape=[
            jax.ShapeDtypeStruct((2, N_NODES, FP), jnp.float32),
            jax.ShapeDtypeStruct((N_NODES, 16), jnp.float32),
        ],
    )(acc, sel, ehpad, bpad, wb0, wb1, wa)


def _final_body(acc_ref, sel_ref, eh_ref, b_ref, wo_ref, bo_ref, out_ref):
    h2 = _normalize(acc_ref, sel_ref, eh_ref, b_ref)
    out_ref[...] = jnp.dot(h2, wo_ref[...],
                           preferred_element_type=jnp.float32) + bo_ref[...]


def _final(acc, sel, ehpad, bpad, wo, bo):
    return pl.pallas_call(
        _final_body,
        grid=(N_NODES // BR,),
        in_specs=[
            pl.BlockSpec((2, BR, FP), lambda i: (0, i, 0)),
            pl.BlockSpec((FP, HEADS), lambda i: (0, 0)),
            pl.BlockSpec((HEADS, F), lambda i: (0, 0)),
            pl.BlockSpec((1, F), lambda i: (0, 0)),
            pl.BlockSpec((F, N_CLASSES), lambda i: (0, 0)),
            pl.BlockSpec((1, N_CLASSES), lambda i: (0, 0)),
        ],
        out_specs=pl.BlockSpec((BR, N_CLASSES), lambda i: (i, 0)),
        out_shape=jax.ShapeDtypeStruct((N_NODES, N_CLASSES), jnp.float32),
    )(acc, sel, ehpad, bpad, wo, bo)


def _edge_body(hs0_hbm, hs1_hbm, atab_hbm, src_hbm, dst_hbm, zeros_hbm,
               out_hbm, acc_sh, src_v, dst_v, msg_v, adst_v, sem_h, sem_a):
    c = lax.axis_index("c")
    s = lax.axis_index("s")
    row0 = s * RPT

    pltpu.sync_copy(zeros_hbm, msg_v)
    for k in range(NZCH):
        pltpu.sync_copy(msg_v, acc_sh.at[pl.ds(row0 + k * CHUNK, CHUNK)])
    plsc.subcore_barrier()

    pltpu.sync_copy(src_hbm.at[s], src_v)
    pltpu.sync_copy(dst_hbm.at[s], dst_v)

    lanes = lax.iota(jnp.int32, 16)
    wmask = lanes < HEADS

    def run(hs_hbm, head_map):
        def chunk_step(j, carry):
            pltpu.async_copy(hs_hbm.at[src_v.at[j]], msg_v, sem_h).wait()
            pltpu.async_copy(atab_hbm.at[dst_v.at[j]], adst_v, sem_a).wait()

            def edge_step(e, carry2):
                a_s = msg_v[e, pl.ds(FH, 16)]
                a_d = adst_v[e, :]
                logit = a_s + a_d
                lrelu = jnp.where(logit >= 0.0, logit, 0.2 * logit)
                w = jnp.where(wmask, jnp.exp(lrelu), 0.0)
                msg_v[e, pl.ds(FH, 16)] = w
                ws = [jnp.full((16,), w[h]) for h in range(HEADS)]
                for j6 in range(FH // 16):
                    msg_v[e, pl.ds(j6 * 16, 16)] = (
                        msg_v[e, pl.ds(j6 * 16, 16)] * ws[head_map[j6]])
                return carry2

            lax.fori_loop(0, CHUNK, edge_step, 0)
            pltpu.sync_copy(msg_v, acc_sh.at[dst_v.at[j]], add=True)
            return carry

        lax.fori_loop(0, NCHUNK, chunk_step, 0)

    @pl.when(c == 0)
    def _():
        run(hs0_hbm, HEAD_MAP[0])

    @pl.when(c == 1)
    def _():
        run(hs1_hbm, HEAD_MAP[1])

    plsc.subcore_barrier()

    for k in range(NZCH):
        pltpu.sync_copy(acc_sh.at[pl.ds(row0 + k * CHUNK, CHUNK)], msg_v)
        pltpu.sync_copy(msg_v, out_hbm.at[c, pl.ds(row0 + k * CHUNK, CHUNK)])


@functools.cache
def _make_edge_pass():
    return functools.partial(
        pl.kernel,
        out_type=jax.ShapeDtypeStruct((NC, NP, FP), jnp.float32),
        mesh=plsc.VectorSubcoreMesh(core_axis_name="c", subcore_axis_name="s",
                                    num_cores=NC, num_subcores=NS),
        compiler_params=pltpu.CompilerParams(use_tc_tiling_on_sc=False),
        scratch_types=[
            pltpu.VMEM_SHARED((NP, FP), jnp.float32),
            pltpu.VMEM((NCHUNK, CHUNK), jnp.int32),
            pltpu.VMEM((NCHUNK, CHUNK), jnp.int32),
            pltpu.VMEM((CHUNK, FP), jnp.float32),
            pltpu.VMEM((CHUNK, 16), jnp.float32),
            pltpu.SemaphoreType.DMA,
            pltpu.SemaphoreType.DMA,
        ],
    )(_edge_body)


def _att_mat(att):
    return (jnp.eye(HEADS, dtype=att.dtype)[:, None, :]
            * att[:, :, None]).reshape(F, HEADS)


def kernel(x, edge_index, W1, att_src1, att_dst1, b1,
           W2, att_src2, att_dst2, b2, Wo, bo):
    f32 = jnp.float32

    def widen(W, att_src):
        k = W.shape[0]
        a = W @ _att_mat(att_src)
        pad = jnp.zeros((k, FP - FH - HEADS), f32)
        return (jnp.concatenate([W[:, :FH], a, pad], axis=1),
                jnp.concatenate([W[:, FH:], a, pad], axis=1))

    def narrow(W, att_dst):
        k = W.shape[0]
        return jnp.concatenate(
            [W @ _att_mat(att_dst), jnp.zeros((k, 16 - HEADS), f32)], axis=1)

    wb1_0, wb1_1 = widen(W1, att_src1)
    wa1 = narrow(W1, att_dst1)
    wb2_0, wb2_1 = widen(W2, att_src2)
    wa2 = narrow(W2, att_dst2)

    sel = (jnp.arange(FP)[:, None] == (FH + jnp.arange(HEADS))[None, :]
           ).astype(f32)
    ehpad = (jnp.arange(F)[None, :] // HID == jnp.arange(HEADS)[:, None]
             ).astype(f32)
    b1p = b1[None, :]
    b2p = b2[None, :]

    loop = jnp.arange(N_NODES, dtype=jnp.int32)
    src = jnp.concatenate([
        edge_index[0].astype(jnp.int32), loop,
        jnp.zeros((EPAD - E_TOT,), jnp.int32)]).reshape(NS, NCHUNK, CHUNK)
    dst = jnp.concatenate([
        edge_index[1].astype(jnp.int32), loop,
        jnp.full((EPAD - E_TOT,), N_NODES, jnp.int32)]).reshape(
            NS, NCHUNK, CHUNK)
    zeros_np = jnp.zeros((CHUNK, FP), f32)

    edge_pass = _make_edge_pass()
    hst1, atab1 = _proj(x, wb1_0, wb1_1, wa1)
    acc1 = edge_pass(hst1[0], hst1[1], atab1, src, dst, zeros_np)
    hst2, atab2 = _combine_proj(acc1, sel, ehpad, b1p, wb2_0, wb2_1, wa2)
    acc2 = edge_pass(hst2[0], hst2[1], atab2, src, dst, zeros_np)
    return _final(acc2, sel, ehpad, b2p, Wo, bo[None, :])


# speedup vs baseline: 45.9937x; 1.5100x over previous
import functools

import jax
import jax.numpy as jnp
from jax import lax
from jax.experimental import pallas as pl
from jax.experimental.pallas import tpu as pltpu
from jax.experimental.pallas import tpu_sc as plsc

N_NODES = 10000
D_IN = 128
HID = 64
HEADS = 3
F = HEADS * HID
FH = 96
FP = FH + 16
N_CLASSES = 40

NC = 2
NS = 16
CHUNK = 64
NP = 10240
RPT = NP // NS
NZCH = RPT // CHUNK

E_TOT = 330000
_NC0 = -(-E_TOT // (NS * CHUNK))
NCHUNK = _NC0 + (_NC0 % 2)
EPAD = NS * NCHUNK * CHUNK

BR = 1000

HEAD_MAP = ((0, 0, 0, 0, 1, 1), (1, 1, 2, 2, 2, 2))


def _proj_body(x_ref, w0_ref, w1_ref, wa_ref, hst_ref, atab_ref):
    x = x_ref[...]
    hst_ref[0] = jnp.dot(x, w0_ref[...], preferred_element_type=jnp.float32)
    hst_ref[1] = jnp.dot(x, w1_ref[...], preferred_element_type=jnp.float32)
    atab_ref[...] = jnp.dot(x, wa_ref[...], preferred_element_type=jnp.float32)


def _proj(x, wb0, wb1, wa):
    k = x.shape[1]
    return pl.pallas_call(
        _proj_body,
        grid=(N_NODES // BR,),
        in_specs=[
            pl.BlockSpec((BR, k), lambda i: (i, 0)),
            pl.BlockSpec((k, FP), lambda i: (0, 0)),
            pl.BlockSpec((k, FP), lambda i: (0, 0)),
            pl.BlockSpec((k, 16), lambda i: (0, 0)),
        ],
        out_specs=[
            pl.BlockSpec((2, BR, FP), lambda i: (0, i, 0)),
            pl.BlockSpec((BR, 16), lambda i: (i, 0)),
        ],
        out_shape=[
            jax.ShapeDtypeStruct((2, N_NODES, FP), jnp.float32),
            jax.ShapeDtypeStruct((N_NODES, 16), jnp.float32),
        ],
    )(x, wb0, wb1, wa)


def _normalize(acc_ref, sel_ref, eh_ref, b_ref):
    val0 = acc_ref[0]
    val1 = acc_ref[1]
    msg = jnp.concatenate([val0[:, :FH], val1[:, :FH]], axis=1)
    den = jnp.dot(val0, sel_ref[...], preferred_element_type=jnp.float32)
    dfull = jnp.dot(1.0 / (den + 1e-16), eh_ref[...],
                    preferred_element_type=jnp.float32)
    return jnp.maximum(msg * dfull + b_ref[...], 0.0)


def _combine_body(acc_ref, sel_ref, eh_ref, b_ref, w0_ref, w1_ref, wa_ref,
                  hst_ref, atab_ref):
    h1 = _normalize(acc_ref, sel_ref, eh_ref, b_ref)
    hst_ref[0] = jnp.dot(h1, w0_ref[...], preferred_element_type=jnp.float32)
    hst_ref[1] = jnp.dot(h1, w1_ref[...], preferred_element_type=jnp.float32)
    atab_ref[...] = jnp.dot(h1, wa_ref[...],
                            preferred_element_type=jnp.float32)


def _combine_proj(acc, sel, ehpad, bpad, wb0, wb1, wa):
    return pl.pallas_call(
        _combine_body,
        grid=(N_NODES // BR,),
        in_specs=[
            pl.BlockSpec((2, BR, FP), lambda i: (0, i, 0)),
            pl.BlockSpec((FP, HEADS), lambda i: (0, 0)),
            pl.BlockSpec((HEADS, F), lambda i: (0, 0)),
            pl.BlockSpec((1, F), lambda i: (0, 0)),
            pl.BlockSpec((F, FP), lambda i: (0, 0)),
            pl.BlockSpec((F, FP), lambda i: (0, 0)),
            pl.BlockSpec((F, 16), lambda i: (0, 0)),
        ],
        out_specs=[
            pl.BlockSpec((2, BR, FP), lambda i: (0, i, 0)),
            pl.BlockSpec((BR, 16), lambda i: (i, 0)),
        ],
        out_shape=[
            jax.ShapeDtypeStruct((2, N_NODES, FP), jnp.float32),
            jax.ShapeDtypeStruct((N_NODES, 16), jnp.float32),
        ],
    )(acc, sel, ehpad, bpad, wb0, wb1, wa)


def _final_body(acc_ref, sel_ref, eh_ref, b_ref, wo_ref, bo_ref, out_ref):
    h2 = _normalize(acc_ref, sel_ref, eh_ref, b_ref)
    out_ref[...] = jnp.dot(h2, wo_ref[...],
                           preferred_element_type=jnp.float32) + bo_ref[...]


def _final(acc, sel, ehpad, bpad, wo, bo):
    return pl.pallas_call(
        _final_body,
        grid=(N_NODES // BR,),
        in_specs=[
            pl.BlockSpec((2, BR, FP), lambda i: (0, i, 0)),
            pl.BlockSpec((FP, HEADS), lambda i: (0, 0)),
            pl.BlockSpec((HEADS, F), lambda i: (0, 0)),
            pl.BlockSpec((1, F), lambda i: (0, 0)),
            pl.BlockSpec((F, N_CLASSES), lambda i: (0, 0)),
            pl.BlockSpec((1, N_CLASSES), lambda i: (0, 0)),
        ],
        out_specs=pl.BlockSpec((BR, N_CLASSES), lambda i: (i, 0)),
        out_shape=jax.ShapeDtypeStruct((N_NODES, N_CLASSES), jnp.float32),
    )(acc, sel, ehpad, bpad, wo, bo)


def _edge_body(hs0_hbm, hs1_hbm, atab_hbm, src_hbm, dst_hbm, zeros_hbm,
               out_hbm, acc_sh, src_v, dst_v, m0, m1, a0, a1,
               sh0, sh1, sa0, sa1):
    c = lax.axis_index("c")
    s = lax.axis_index("s")
    row0 = s * RPT

    pltpu.sync_copy(zeros_hbm, m0)
    for k in range(NZCH):
        pltpu.sync_copy(m0, acc_sh.at[pl.ds(row0 + k * CHUNK, CHUNK)])
    plsc.subcore_barrier()

    pltpu.sync_copy(src_hbm.at[s], src_v)
    pltpu.sync_copy(dst_hbm.at[s], dst_v)

    lanes = lax.iota(jnp.int32, 16)
    wmask = lanes < HEADS

    def run(hs_hbm, head_map):
        def start(j, mb, ab, sh, sa):
            pltpu.async_copy(hs_hbm.at[src_v.at[j]], mb, sh)
            pltpu.async_copy(atab_hbm.at[dst_v.at[j]], ab, sa)

        def wait(j, mb, ab, sh, sa):
            pltpu.make_async_copy(hs_hbm.at[src_v.at[j]], mb, sh).wait()
            pltpu.make_async_copy(atab_hbm.at[dst_v.at[j]], ab, sa).wait()

        def process(j, mb, ab):
            def edge_step(e, carry2):
                a_s = mb[e, pl.ds(FH, 16)]
                a_d = ab[e, :]
                logit = a_s + a_d
                lrelu = jnp.where(logit >= 0.0, logit, 0.2 * logit)
                w = jnp.where(wmask, jnp.exp(lrelu), 0.0)
                mb[e, pl.ds(FH, 16)] = w
                ws = [jnp.full((16,), w[h]) for h in range(HEADS)]
                for j6 in range(FH // 16):
                    mb[e, pl.ds(j6 * 16, 16)] = (
                        mb[e, pl.ds(j6 * 16, 16)] * ws[head_map[j6]])
                return carry2

            lax.fori_loop(0, CHUNK, edge_step, 0)
            pltpu.sync_copy(mb, acc_sh.at[dst_v.at[j]], add=True)

        start(0, m0, a0, sh0, sa0)

        def pair_step(jj, carry):
            j0 = 2 * jj
            j1 = j0 + 1
            wait(j0, m0, a0, sh0, sa0)
            start(j1, m1, a1, sh1, sa1)
            process(j0, m0, a0)
            wait(j1, m1, a1, sh1, sa1)

            @pl.when(j1 + 1 < NCHUNK)
            def _():
                start(j1 + 1, m0, a0, sh0, sa0)

            process(j1, m1, a1)
            return carry

        lax.fori_loop(0, NCHUNK // 2, pair_step, 0)

    @pl.when(c == 0)
    def _():
        run(hs0_hbm, HEAD_MAP[0])

    @pl.when(c == 1)
    def _():
        run(hs1_hbm, HEAD_MAP[1])

    plsc.subcore_barrier()

    for k in range(NZCH):
        pltpu.sync_copy(acc_sh.at[pl.ds(row0 + k * CHUNK, CHUNK)], m0)
        pltpu.sync_copy(m0, out_hbm.at[c, pl.ds(row0 + k * CHUNK, CHUNK)])


@functools.cache
def _make_edge_pass():
    return functools.partial(
        pl.kernel,
        out_type=jax.ShapeDtypeStruct((NC, NP, FP), jnp.float32),
        mesh=plsc.VectorSubcoreMesh(core_axis_name="c", subcore_axis_name="s",
                                    num_cores=NC, num_subcores=NS),
        compiler_params=pltpu.CompilerParams(use_tc_tiling_on_sc=False),
        scratch_types=[
            pltpu.VMEM_SHARED((NP, FP), jnp.float32),
            pltpu.VMEM((NCHUNK, CHUNK), jnp.int32),
            pltpu.VMEM((NCHUNK, CHUNK), jnp.int32),
            pltpu.VMEM((CHUNK, FP), jnp.float32),
            pltpu.VMEM((CHUNK, FP), jnp.float32),
            pltpu.VMEM((CHUNK, 16), jnp.float32),
            pltpu.VMEM((CHUNK, 16), jnp.float32),
            pltpu.SemaphoreType.DMA,
            pltpu.SemaphoreType.DMA,
            pltpu.SemaphoreType.DMA,
            pltpu.SemaphoreType.DMA,
        ],
    )(_edge_body)


def _att_mat(att):
    return (jnp.eye(HEADS, dtype=att.dtype)[:, None, :]
            * att[:, :, None]).reshape(F, HEADS)


def kernel(x, edge_index, W1, att_src1, att_dst1, b1,
           W2, att_src2, att_dst2, b2, Wo, bo):
    f32 = jnp.float32

    def widen(W, att_src):
        k = W.shape[0]
        a = W @ _att_mat(att_src)
        pad = jnp.zeros((k, FP - FH - HEADS), f32)
        return (jnp.concatenate([W[:, :FH], a, pad], axis=1),
                jnp.concatenate([W[:, FH:], a, pad], axis=1))

    def narrow(W, att_dst):
        k = W.shape[0]
        return jnp.concatenate(
            [W @ _att_mat(att_dst), jnp.zeros((k, 16 - HEADS), f32)], axis=1)

    wb1_0, wb1_1 = widen(W1, att_src1)
    wa1 = narrow(W1, att_dst1)
    wb2_0, wb2_1 = widen(W2, att_src2)
    wa2 = narrow(W2, att_dst2)

    sel = (jnp.arange(FP)[:, None] == (FH + jnp.arange(HEADS))[None, :]
           ).astype(f32)
    ehpad = (jnp.arange(F)[None, :] // HID == jnp.arange(HEADS)[:, None]
             ).astype(f32)
    b1p = b1[None, :]
    b2p = b2[None, :]

    loop = jnp.arange(N_NODES, dtype=jnp.int32)
    src = jnp.concatenate([
        edge_index[0].astype(jnp.int32), loop,
        jnp.zeros((EPAD - E_TOT,), jnp.int32)]).reshape(NS, NCHUNK, CHUNK)
    dst = jnp.concatenate([
        edge_index[1].astype(jnp.int32), loop,
        jnp.full((EPAD - E_TOT,), N_NODES, jnp.int32)]).reshape(
            NS, NCHUNK, CHUNK)
    zeros_np = jnp.zeros((CHUNK, FP), f32)

    edge_pass = _make_edge_pass()
    hst1, atab1 = _proj(x, wb1_0, wb1_1, wa1)
    acc1 = edge_pass(hst1[0], hst1[1], atab1, src, dst, zeros_np)
    hst2, atab2 = _combine_proj(acc1, sel, ehpad, b1p, wb2_0, wb2_1, wa2)
    acc2 = edge_pass(hst2[0], hst2[1], atab2, src, dst, zeros_np)
    return _final(acc2, sel, ehpad, b2p, Wo, bo[None, :])
